# Initial kernel scaffold; baseline (speedup 1.0000x reference)
import jax
import jax.numpy as jnp
from jax.experimental import pallas as pl


def kernel(x, edge_index, batch, w1l, b1l, w1r, w2l, b2l, w2r, w3l, b3l, w3r, w4l, b4l, w4r, p1w, p1b, p1r, p2w, p2b, p2r, f1w, f1b, f2w, f2b):
    raise NotImplementedError("write your pallas kernel here")


# speedup vs baseline: 1.0055x; 1.0055x over previous
import functools

import jax
import jax.numpy as jnp
from jax import lax
from jax.experimental import pallas as pl
from jax.experimental.pallas import tpu as pltpu

N = 10000
E = 320000
D = 128
H = 128
B = 64


def _readout_body(h_ref, mask_ref, batch_ref, f1w_ref, f1b_ref, f2w_ref, f2b_ref, out_ref):
    h = h_ref[...]
    m = mask_ref[...]
    bvec = batch_ref[...]
    onehot = jnp.where(
        jnp.equal(bvec, lax.broadcasted_iota(jnp.int32, (B, N), 0)),
        m, jnp.zeros((B, N), jnp.float32))
    s = jnp.dot(onehot, h, preferred_element_type=jnp.float32)
    cnt = jnp.sum(onehot, axis=1, keepdims=True)
    g = s / jnp.maximum(cnt, 1.0)
    z1 = jnp.maximum(
        jnp.dot(g, f1w_ref[...].T, preferred_element_type=jnp.float32)
        + f1b_ref[...], 0.0)
    z = (jnp.dot(z1, f2w_ref[...].T, preferred_element_type=jnp.float32)
         + f2b_ref[...])
    zmax = jnp.max(z, axis=1, keepdims=True)
    lse = jnp.log(jnp.sum(jnp.exp(z - zmax), axis=1, keepdims=True)) + zmax
    out_ref[...] = z - lse


def _readout(h, mask_n, batch, f1w, f1b, f2w, f2b):
    return pl.pallas_call(
        _readout_body,
        out_shape=jax.ShapeDtypeStruct((B, 2), jnp.float32),
    )(h, mask_n.reshape(1, N), batch.reshape(1, N),
      f1w, f1b.reshape(1, 64), f2w, f2b.reshape(1, 2))


def _sage(x, src, dst, mask_e, Wl, bl, Wr):
    msg = x[src] * mask_e[:, None]
    s = jax.ops.segment_sum(msg, dst, num_segments=x.shape[0])
    deg = jax.ops.segment_sum(mask_e, dst, num_segments=x.shape[0])
    agg = s / jnp.clip(deg, 1.0)[:, None]
    return agg @ Wl.T + bl + x @ Wr.T


def _pool(x, src, dst, mask_e, mask_n, batch, Wrel, brel, Wroot, ratio=0.5):
    msg = x[src] * mask_e[:, None]
    aggr = jax.ops.segment_sum(msg, dst, num_segments=x.shape[0])
    score = jnp.tanh((aggr @ Wrel.T + brel + x @ Wroot.T)[:, 0])
    score_m = jnp.where(mask_n > 0, score, -2.0)
    counts = jax.ops.segment_sum(mask_n, batch, num_segments=B)
    k_per = jnp.ceil(ratio * counts).astype(jnp.int32)
    key_sort = batch.astype(jnp.float32) * 8.0 - score_m
    order = jnp.argsort(key_sort)
    sizes = jnp.bincount(batch, length=B)
    starts = jnp.concatenate([jnp.zeros((1,), sizes.dtype), jnp.cumsum(sizes)[:-1]])
    bs = batch[order]
    rank = jnp.arange(x.shape[0]) - starts[bs]
    keep_sorted = rank < k_per[bs]
    keep = jnp.zeros((x.shape[0],), bool).at[order].set(keep_sorted)
    keep_f = keep.astype(jnp.float32) * mask_n
    x_new = x * score[:, None] * keep_f[:, None]
    mask_e_new = mask_e * keep_f[src] * keep_f[dst]
    return x_new, mask_e_new, keep_f


def kernel(x, edge_index, batch, w1l, b1l, w1r, w2l, b2l, w2r, w3l, b3l, w3r,
           w4l, b4l, w4r, p1w, p1b, p1r, p2w, p2b, p2r, f1w, f1b, f2w, f2b):
    src = edge_index[0]
    dst = edge_index[1]
    mask_e = jnp.ones((E,), jnp.float32)
    mask_n = jnp.ones((N,), jnp.float32)
    h = jax.nn.relu(_sage(x, src, dst, mask_e, w1l, b1l, w1r))
    h = jax.nn.relu(_sage(h, src, dst, mask_e, w2l, b2l, w2r))
    h, mask_e, mask_n = _pool(h, src, dst, mask_e, mask_n, batch, p1w, p1b, p1r)
    h = jax.nn.relu(_sage(h, src, dst, mask_e, w3l, b3l, w3r))
    h = jax.nn.relu(_sage(h, src, dst, mask_e, w4l, b4l, w4r))
    h, mask_e, mask_n = _pool(h, src, dst, mask_e, mask_n, batch, p2w, p2b, p2r)
    return _readout(h, mask_n, batch, f1w, f1b, f2w, f2b)


# speedup vs baseline: 4.7361x; 4.7102x over previous
import functools

import jax
import jax.numpy as jnp
from jax import lax
from jax.experimental import pallas as pl
from jax.experimental.pallas import tpu as pltpu
from jax.experimental.pallas import tpu_sc as plsc

N = 10000
E = 320000
H = 128
B = 64

NC = 2
NS = 16
CH = 128
NCHUNK = E // CH
CPS = 80
NCHUNK_PAD = NC * NS * CPS
N_PAD = 10240
RPS = N_PAD // NS


def _edge_pass_body(x_hbm, srcT, dstT, sval_hbm, zmat, zvec,
                    feat_out, scal_out,
                    srcbuf, dstbuf, rows, svbuf, acc, sacc):
    c = lax.axis_index("c")
    s = lax.axis_index("s")
    r0 = s * RPS
    pltpu.sync_copy(zmat.at[pl.ds(r0, RPS)], acc.at[pl.ds(r0, RPS)])
    pltpu.sync_copy(zvec.at[pl.ds(r0, RPS)], sacc.at[pl.ds(r0, RPS)])
    plsc.subcore_barrier()
    base = (c * NS + s) * CPS
    pltpu.sync_copy(srcT.at[pl.ds(base, CPS)], srcbuf)
    pltpu.sync_copy(dstT.at[pl.ds(base, CPS)], dstbuf)

    def body(j, carry):
        pltpu.sync_copy(x_hbm.at[srcbuf.at[j]], rows)
        pltpu.sync_copy(sval_hbm.at[srcbuf.at[j]], svbuf)
        pltpu.sync_copy(rows, acc.at[dstbuf.at[j]], add=True)
        pltpu.sync_copy(svbuf, sacc.at[dstbuf.at[j]], add=True)
        return carry

    lax.fori_loop(0, CPS, body, 0)
    plsc.subcore_barrier()
    pltpu.sync_copy(acc.at[pl.ds(r0, RPS)], feat_out.at[c, pl.ds(r0, RPS)])
    pltpu.sync_copy(sacc.at[pl.ds(r0, RPS)], scal_out.at[c, pl.ds(r0, RPS)])


def _sc_edge_pass(xmat, sval, srcT, dstT, zmat, zvec):
    mesh = plsc.VectorSubcoreMesh(core_axis_name="c", subcore_axis_name="s")
    f = pl.kernel(
        _edge_pass_body,
        out_type=(jax.ShapeDtypeStruct((NC, N_PAD, H), jnp.float32),
                  jax.ShapeDtypeStruct((NC, N_PAD), jnp.float32)),
        mesh=mesh,
        scratch_types=[
            pltpu.VMEM((CPS, CH), jnp.int32),
            pltpu.VMEM((CPS, CH), jnp.int32),
            pltpu.VMEM((CH, H), jnp.float32),
            pltpu.VMEM((CH,), jnp.float32),
            pltpu.VMEM_SHARED((N_PAD, H), jnp.float32),
            pltpu.VMEM_SHARED((N_PAD,), jnp.float32),
        ],
    )
    return f(xmat, srcT, dstT, sval, zmat, zvec)


def _scalar_pass_body(sval_hbm, srcT, dstT, zvec,
                      scal_out,
                      srcbuf, dstbuf, svbuf, sacc):
    c = lax.axis_index("c")
    s = lax.axis_index("s")
    r0 = s * RPS
    pltpu.sync_copy(zvec.at[pl.ds(r0, RPS)], sacc.at[pl.ds(r0, RPS)])
    plsc.subcore_barrier()
    base = (c * NS + s) * CPS
    pltpu.sync_copy(srcT.at[pl.ds(base, CPS)], srcbuf)
    pltpu.sync_copy(dstT.at[pl.ds(base, CPS)], dstbuf)

    def body(j, carry):
        pltpu.sync_copy(sval_hbm.at[srcbuf.at[j]], svbuf)
        pltpu.sync_copy(svbuf, sacc.at[dstbuf.at[j]], add=True)
        return carry

    lax.fori_loop(0, CPS, body, 0)
    plsc.subcore_barrier()
    pltpu.sync_copy(sacc.at[pl.ds(r0, RPS)], scal_out.at[c, pl.ds(r0, RPS)])


def _sc_scalar_pass(sval, srcT, dstT, zvec):
    mesh = plsc.VectorSubcoreMesh(core_axis_name="c", subcore_axis_name="s")
    f = pl.kernel(
        _scalar_pass_body,
        out_type=jax.ShapeDtypeStruct((NC, N_PAD), jnp.float32),
        mesh=mesh,
        scratch_types=[
            pltpu.VMEM((CPS, CH), jnp.int32),
            pltpu.VMEM((CPS, CH), jnp.int32),
            pltpu.VMEM((CH,), jnp.float32),
            pltpu.VMEM_SHARED((N_PAD,), jnp.float32),
        ],
    )
    return f(sval, srcT, dstT, zvec)


def _readout_body(h_ref, m_ref, k_ref, batch_ref, f1w_ref, f1b_ref, f2w_ref,
                  f2b_ref, out_ref):
    h = h_ref[...]
    m = m_ref[...]
    kf = k_ref[...]
    bvec = batch_ref[...]
    oh = jnp.equal(bvec, lax.broadcasted_iota(jnp.int32, (B, N), 0))
    wvals = jnp.where(oh, m, jnp.zeros((B, N), jnp.float32))
    s = jnp.dot(wvals, h, preferred_element_type=jnp.float32)
    cnt = jnp.sum(jnp.where(oh, kf, jnp.zeros((B, N), jnp.float32)),
                  axis=1, keepdims=True)
    g = s / jnp.maximum(cnt, 1.0)
    z1 = jnp.maximum(
        lax.dot_general(g, f1w_ref[...], (((1,), (1,)), ((), ())),
                        preferred_element_type=jnp.float32)
        + f1b_ref[...], 0.0)
    z = (lax.dot_general(z1, f2w_ref[...], (((1,), (1,)), ((), ())),
                         preferred_element_type=jnp.float32)
         + f2b_ref[...])
    zmax = jnp.max(z, axis=1, keepdims=True)
    lse = jnp.log(jnp.sum(jnp.exp(z - zmax), axis=1, keepdims=True)) + zmax
    out_ref[...] = z - lse


def _readout(h, m, keep, batch, f1w, f1b, f2w, f2b):
    return pl.pallas_call(
        _readout_body,
        out_shape=jax.ShapeDtypeStruct((B, 2), jnp.float32),
    )(h, m.reshape(1, N), keep.reshape(1, N), batch.reshape(1, N),
      f1w, f1b.reshape(1, 64), f2w, f2b.reshape(1, 2))


def _keep_from_scores(score_m, batch, counts):
    k_per = jnp.ceil(0.5 * counts).astype(jnp.int32)
    key = batch.astype(jnp.float32) * 8.0 - score_m
    order = jnp.argsort(key)
    sizes = jnp.bincount(batch, length=B)
    starts = jnp.concatenate([jnp.zeros((1,), sizes.dtype),
                              jnp.cumsum(sizes)[:-1]])
    bs = batch[order]
    rank = jnp.arange(N) - starts[bs]
    keep_sorted = rank < k_per[bs]
    keep = jnp.zeros((N,), bool).at[order].set(keep_sorted)
    return keep.astype(jnp.float32)


def kernel(x, edge_index, batch, w1l, b1l, w1r, w2l, b2l, w2r, w3l, b3l, w3r,
           w4l, b4l, w4r, p1w, p1b, p1r, p2w, p2b, p2r, f1w, f1b, f2w, f2b):
    src = edge_index[0]
    dst = edge_index[1]
    pad_e = (NCHUNK_PAD - NCHUNK) * CH
    srcT = jnp.concatenate(
        [src, jnp.zeros((pad_e,), jnp.int32)]).reshape(NCHUNK_PAD, CH)
    dstT = jnp.concatenate(
        [dst, N + (jnp.arange(pad_e, dtype=jnp.int32) % (N_PAD - N))]
    ).reshape(NCHUNK_PAD, CH)
    zmat = jnp.zeros((N_PAD, H), jnp.float32)
    zvec = jnp.zeros((N_PAD,), jnp.float32)
    ones_n = jnp.ones((N,), jnp.float32)

    P1, degp = _sc_edge_pass(x, ones_n, srcT, dstT, zmat, zvec)
    deg = jnp.clip(degp[0, :N] + degp[1, :N], 1.0)[:, None]
    h1 = jax.nn.relu((P1[0, :N] + P1[1, :N]) / deg @ w1l.T + b1l + x @ w1r.T)
    P2, _ = _sc_edge_pass(h1, ones_n, srcT, dstT, zmat, zvec)
    h2 = jax.nn.relu((P2[0, :N] + P2[1, :N]) / deg @ w2l.T + b2l + h1 @ w2r.T)
    a1 = (h2 @ p1w.T)[:, 0]
    c1 = (h2 @ p1r.T)[:, 0]
    sp1p = _sc_scalar_pass(a1, srcT, dstT, zvec)
    score1 = jnp.tanh(sp1p[0, :N] + sp1p[1, :N] + p1b[0] + c1)
    keep1 = _keep_from_scores(score1, batch, jnp.full((B,), 0.0)
                              + jnp.bincount(batch, length=B).astype(jnp.float32))
    m1 = score1 * keep1
    x3 = h2 * m1[:, None]
    P3, kdegp = _sc_edge_pass(x3, keep1, srcT, dstT, zmat, zvec)
    kdeg = jnp.clip(kdegp[0, :N] + kdegp[1, :N], 1.0)[:, None]
    h3 = jax.nn.relu(
        (keep1[:, None] * (P3[0, :N] + P3[1, :N])) / kdeg @ w3l.T + b3l
        + x3 @ w3r.T
    ) * keep1[:, None]
    P4, _ = _sc_edge_pass(h3, ones_n, srcT, dstT, zmat, zvec)
    h4 = jax.nn.relu(
        (keep1[:, None] * (P4[0, :N] + P4[1, :N])) / kdeg @ w4l.T + b4l
        + h3 @ w4r.T)
    az = (h4 @ p2w.T)[:, 0] * keep1
    c2 = (h4 @ p2r.T)[:, 0]
    sp2p = _sc_scalar_pass(az, srcT, dstT, zvec)
    score2 = jnp.tanh(keep1 * (sp2p[0, :N] + sp2p[1, :N]) + p2b[0] + c2)
    score_m2 = jnp.where(keep1 > 0, score2, -2.0)
    counts2 = jax.ops.segment_sum(keep1, batch, num_segments=B)
    keep2 = _keep_from_scores(score_m2, batch, counts2) * keep1
    m2 = score_m2 * keep2
    return _readout(h4, m2, keep2, batch, f1w, f1b, f2w, f2b)


# speedup vs baseline: 5.1187x; 1.0808x over previous
import functools

import jax
import jax.numpy as jnp
from jax import lax
from jax.experimental import pallas as pl
from jax.experimental.pallas import tpu as pltpu
from jax.experimental.pallas import tpu_sc as plsc

N = 10000
E = 320000
H = 128
B = 64

NC = 2
NS = 16
CH = 128
NCHUNK = E // CH
CPS = 80
NCHUNK_PAD = NC * NS * CPS
N_PAD = 10240
RPS = N_PAD // NS


NBUF = 2
NG = CPS // NBUF
NBUF_S = 8
NG_S = CPS // NBUF_S


def _unpack_idx(pkbuf, j, ring, b, shift):
    for i in range(CH // 16):
        v = pkbuf[j, pl.ds(i * 16, 16)]
        if shift:
            w = lax.shift_right_logical(v, 16)
        else:
            w = lax.bitwise_and(v, jnp.full((16,), 0xFFFF, jnp.int32))
        ring[b, pl.ds(i * 16, 16)] = w


def _edge_pass_body(mode, x_hbm, pkT, sval_hbm, zmat, zvec,
                    feat_out, scal_out,
                    pkbuf, srcring, dstring, rows, svbuf, onesbuf, acc, sacc,
                    gsem, ssem, sgsem, sssem):
    c = lax.axis_index("c")
    s = lax.axis_index("s")
    r0 = s * RPS
    pltpu.sync_copy(zmat.at[pl.ds(r0, RPS)], acc.at[pl.ds(r0, RPS)])
    if mode:
        pltpu.sync_copy(zvec.at[pl.ds(r0, RPS)], sacc.at[pl.ds(r0, RPS)])
    if mode == 1:
        for i in range(CH // 16):
            onesbuf[pl.ds(i * 16, 16)] = jnp.ones((16,), jnp.float32)
    plsc.subcore_barrier()
    base = (c * NS + s) * CPS
    pltpu.sync_copy(pkT.at[pl.ds(base, CPS)], pkbuf)

    def g_desc(b):
        return pltpu.make_async_copy(
            x_hbm.at[srcring.at[b]], rows.at[b], gsem.at[b])

    def s_desc(b):
        return pltpu.make_async_copy(
            rows.at[b], acc.at[dstring.at[b]], ssem.at[b])

    def sg_desc(b):
        return pltpu.make_async_copy(
            sval_hbm.at[srcring.at[b]], svbuf.at[b], sgsem.at[b])

    def ss_desc(b):
        src = onesbuf if mode == 1 else svbuf.at[b]
        return pltpu.make_async_copy(
            src, sacc.at[dstring.at[b]], sssem.at[b])

    for b in range(NBUF):
        _unpack_idx(pkbuf, b, srcring, b, 0)
        _unpack_idx(pkbuf, b, dstring, b, 1)
        g_desc(b).start()
        if mode == 2:
            sg_desc(b).start()

    def g_body(g, carry):
        j0 = g * NBUF
        for b in range(NBUF):
            g_desc(b).wait()
            if mode == 2:
                sg_desc(b).wait()
        for b in range(NBUF):
            pltpu.async_copy(rows.at[b], acc.at[dstring.at[b]],
                             ssem.at[b], add=True)
            if mode:
                src = onesbuf if mode == 1 else svbuf.at[b]
                pltpu.async_copy(src, sacc.at[dstring.at[b]],
                                 sssem.at[b], add=True)

        @pl.when(g < NG - 1)
        def _():
            for b in range(NBUF):
                _unpack_idx(pkbuf, j0 + NBUF + b, srcring, b, 0)
        for b in range(NBUF):
            s_desc(b).wait()
            if mode:
                ss_desc(b).wait()

        @pl.when(g < NG - 1)
        def _():
            for b in range(NBUF):
                _unpack_idx(pkbuf, j0 + NBUF + b, dstring, b, 1)
                g_desc(b).start()
                if mode == 2:
                    sg_desc(b).start()
        return carry

    lax.fori_loop(0, NG, g_body, 0)
    plsc.subcore_barrier()
    pltpu.sync_copy(acc.at[pl.ds(r0, RPS)], feat_out.at[c, pl.ds(r0, RPS)])
    if mode:
        pltpu.sync_copy(sacc.at[pl.ds(r0, RPS)],
                        scal_out.at[c, pl.ds(r0, RPS)])
    else:
        pltpu.sync_copy(zvec.at[pl.ds(r0, RPS)],
                        scal_out.at[c, pl.ds(r0, RPS)])


def _sc_edge_pass(xmat, sval, pkT, zmat, zvec, mode):
    mesh = plsc.VectorSubcoreMesh(core_axis_name="c", subcore_axis_name="s")
    f = pl.kernel(
        functools.partial(_edge_pass_body, mode),
        out_type=(jax.ShapeDtypeStruct((NC, N_PAD, H), jnp.float32),
                  jax.ShapeDtypeStruct((NC, N_PAD), jnp.float32)),
        mesh=mesh,
        scratch_types=[
            pltpu.VMEM((CPS, CH), jnp.int32),
            pltpu.VMEM((NBUF, CH), jnp.int32),
            pltpu.VMEM((NBUF, CH), jnp.int32),
            pltpu.VMEM((NBUF, CH, H), jnp.float32),
            pltpu.VMEM((NBUF, CH), jnp.float32),
            pltpu.VMEM((CH,), jnp.float32),
            pltpu.VMEM_SHARED((N_PAD, H), jnp.float32),
            pltpu.VMEM_SHARED((N_PAD,), jnp.float32),
            pltpu.SemaphoreType.DMA((NBUF,)),
            pltpu.SemaphoreType.DMA((NBUF,)),
            pltpu.SemaphoreType.DMA((NBUF,)),
            pltpu.SemaphoreType.DMA((NBUF,)),
        ],
    )
    return f(xmat, pkT, sval, zmat, zvec)


def _scalar_pass_body(sval_hbm, pkT, zvec,
                      scal_out,
                      pkbuf, srcring, dstring, svbuf, sacc, gsem, ssem):
    c = lax.axis_index("c")
    s = lax.axis_index("s")
    r0 = s * RPS
    pltpu.sync_copy(zvec.at[pl.ds(r0, RPS)], sacc.at[pl.ds(r0, RPS)])
    plsc.subcore_barrier()
    base = (c * NS + s) * CPS
    pltpu.sync_copy(pkT.at[pl.ds(base, CPS)], pkbuf)

    def g_desc(b):
        return pltpu.make_async_copy(
            sval_hbm.at[srcring.at[b]], svbuf.at[b], gsem.at[b])

    def s_desc(b):
        return pltpu.make_async_copy(
            svbuf.at[b], sacc.at[dstring.at[b]], ssem.at[b])

    for b in range(NBUF_S):
        _unpack_idx(pkbuf, b, srcring, b, 0)
        _unpack_idx(pkbuf, b, dstring, b, 1)
        g_desc(b).start()

    def g_body(g, carry):
        j0 = g * NBUF_S
        for b in range(NBUF_S):
            g_desc(b).wait()
        for b in range(NBUF_S):
            pltpu.async_copy(svbuf.at[b], sacc.at[dstring.at[b]],
                             ssem.at[b], add=True)

        @pl.when(g < NG_S - 1)
        def _():
            for b in range(NBUF_S):
                _unpack_idx(pkbuf, j0 + NBUF_S + b, srcring, b, 0)
        for b in range(NBUF_S):
            s_desc(b).wait()

        @pl.when(g < NG_S - 1)
        def _():
            for b in range(NBUF_S):
                _unpack_idx(pkbuf, j0 + NBUF_S + b, dstring, b, 1)
                g_desc(b).start()
        return carry

    lax.fori_loop(0, NG_S, g_body, 0)
    plsc.subcore_barrier()
    pltpu.sync_copy(sacc.at[pl.ds(r0, RPS)], scal_out.at[c, pl.ds(r0, RPS)])


def _sc_scalar_pass(sval, pkT, zvec):
    mesh = plsc.VectorSubcoreMesh(core_axis_name="c", subcore_axis_name="s")
    f = pl.kernel(
        _scalar_pass_body,
        out_type=jax.ShapeDtypeStruct((NC, N_PAD), jnp.float32),
        mesh=mesh,
        scratch_types=[
            pltpu.VMEM((CPS, CH), jnp.int32),
            pltpu.VMEM((NBUF_S, CH), jnp.int32),
            pltpu.VMEM((NBUF_S, CH), jnp.int32),
            pltpu.VMEM((NBUF_S, CH), jnp.float32),
            pltpu.VMEM_SHARED((N_PAD,), jnp.float32),
            pltpu.SemaphoreType.DMA((NBUF_S,)),
            pltpu.SemaphoreType.DMA((NBUF_S,)),
        ],
    )
    return f(sval, pkT, zvec)


def _readout_body(h_ref, m_ref, k_ref, batch_ref, f1w_ref, f1b_ref, f2w_ref,
                  f2b_ref, out_ref):
    h = h_ref[...]
    m = m_ref[...]
    kf = k_ref[...]
    bvec = batch_ref[...]
    oh = jnp.equal(bvec, lax.broadcasted_iota(jnp.int32, (B, N), 0))
    wvals = jnp.where(oh, m, jnp.zeros((B, N), jnp.float32))
    s = jnp.dot(wvals, h, preferred_element_type=jnp.float32)
    cnt = jnp.sum(jnp.where(oh, kf, jnp.zeros((B, N), jnp.float32)),
                  axis=1, keepdims=True)
    g = s / jnp.maximum(cnt, 1.0)
    z1 = jnp.maximum(
        lax.dot_general(g, f1w_ref[...], (((1,), (1,)), ((), ())),
                        preferred_element_type=jnp.float32)
        + f1b_ref[...], 0.0)
    z = (lax.dot_general(z1, f2w_ref[...], (((1,), (1,)), ((), ())),
                         preferred_element_type=jnp.float32)
         + f2b_ref[...])
    zmax = jnp.max(z, axis=1, keepdims=True)
    lse = jnp.log(jnp.sum(jnp.exp(z - zmax), axis=1, keepdims=True)) + zmax
    out_ref[...] = z - lse


def _readout(h, m, keep, batch, f1w, f1b, f2w, f2b):
    return pl.pallas_call(
        _readout_body,
        out_shape=jax.ShapeDtypeStruct((B, 2), jnp.float32),
    )(h, m.reshape(1, N), keep.reshape(1, N), batch.reshape(1, N),
      f1w, f1b.reshape(1, 64), f2w, f2b.reshape(1, 2))


def _keep_from_scores(score_m, batch, counts):
    k_per = jnp.ceil(0.5 * counts).astype(jnp.int32)
    key = batch.astype(jnp.float32) * 8.0 - score_m
    order = jnp.argsort(key)
    sizes = jnp.bincount(batch, length=B)
    starts = jnp.concatenate([jnp.zeros((1,), sizes.dtype),
                              jnp.cumsum(sizes)[:-1]])
    bs = batch[order]
    rank = jnp.arange(N) - starts[bs]
    keep_sorted = rank < k_per[bs]
    keep = jnp.zeros((N,), bool).at[order].set(keep_sorted)
    return keep.astype(jnp.float32)


def kernel(x, edge_index, batch, w1l, b1l, w1r, w2l, b2l, w2r, w3l, b3l, w3r,
           w4l, b4l, w4r, p1w, p1b, p1r, p2w, p2b, p2r, f1w, f1b, f2w, f2b):
    src = edge_index[0]
    dst = edge_index[1]
    pad_e = (NCHUNK_PAD - NCHUNK) * CH
    src_p = jnp.concatenate([src, jnp.zeros((pad_e,), jnp.int32)])
    dst_p = jnp.concatenate(
        [dst, N + (jnp.arange(pad_e, dtype=jnp.int32) % (N_PAD - N))])
    pkT = jnp.bitwise_or(jnp.left_shift(dst_p, 16),
                         src_p).reshape(NCHUNK_PAD, CH)
    zmat = jnp.zeros((N_PAD, H), jnp.float32)
    zvec = jnp.zeros((N_PAD,), jnp.float32)
    ones_n = jnp.ones((N,), jnp.float32)

    P1, degp = _sc_edge_pass(x, ones_n, pkT, zmat, zvec, 1)
    deg = jnp.clip(degp[0, :N] + degp[1, :N], 1.0)[:, None]
    h1 = jax.nn.relu((P1[0, :N] + P1[1, :N]) / deg @ w1l.T + b1l + x @ w1r.T)
    P2, _ = _sc_edge_pass(h1, ones_n, pkT, zmat, zvec, 0)
    h2 = jax.nn.relu((P2[0, :N] + P2[1, :N]) / deg @ w2l.T + b2l + h1 @ w2r.T)
    a1 = (h2 @ p1w.T)[:, 0]
    c1 = (h2 @ p1r.T)[:, 0]
    sp1p = _sc_scalar_pass(a1, pkT, zvec)
    score1 = jnp.tanh(sp1p[0, :N] + sp1p[1, :N] + p1b[0] + c1)
    keep1 = _keep_from_scores(score1, batch, jnp.full((B,), 0.0)
                              + jnp.bincount(batch, length=B).astype(jnp.float32))
    m1 = score1 * keep1
    x3 = h2 * m1[:, None]
    P3, kdegp = _sc_edge_pass(x3, keep1, pkT, zmat, zvec, 2)
    kdeg = jnp.clip(kdegp[0, :N] + kdegp[1, :N], 1.0)[:, None]
    h3 = jax.nn.relu(
        (keep1[:, None] * (P3[0, :N] + P3[1, :N])) / kdeg @ w3l.T + b3l
        + x3 @ w3r.T
    ) * keep1[:, None]
    P4, _ = _sc_edge_pass(h3, ones_n, pkT, zmat, zvec, 0)
    h4 = jax.nn.relu(
        (keep1[:, None] * (P4[0, :N] + P4[1, :N])) / kdeg @ w4l.T + b4l
        + h3 @ w4r.T)
    az = (h4 @ p2w.T)[:, 0] * keep1
    c2 = (h4 @ p2r.T)[:, 0]
    sp2p = _sc_scalar_pass(az, pkT, zvec)
    score2 = jnp.tanh(keep1 * (sp2p[0, :N] + sp2p[1, :N]) + p2b[0] + c2)
    score_m2 = jnp.where(keep1 > 0, score2, -2.0)
    counts2 = jax.ops.segment_sum(keep1, batch, num_segments=B)
    keep2 = _keep_from_scores(score_m2, batch, counts2) * keep1
    m2 = score_m2 * keep2
    return _readout(h4, m2, keep2, batch, f1w, f1b, f2w, f2b)


# speedup vs baseline: 5.4527x; 1.0653x over previous
import functools

import jax
import jax.numpy as jnp
from jax import lax
from jax.experimental import pallas as pl
from jax.experimental.pallas import tpu as pltpu
from jax.experimental.pallas import tpu_sc as plsc

N = 10000
E = 320000
H = 128
B = 64

NC = 2
NS = 16
CH = 128
NCHUNK = E // CH
CPS = 80
NCHUNK_PAD = NC * NS * CPS
N_PAD = 10240
RPS = N_PAD // NS


NBUF = 2
NG = CPS // NBUF
NBUF_S = 8
NG_S = CPS // NBUF_S


def _unpack_idx(pkbuf, j, ring, b, shift):
    for i in range(CH // 16):
        v = pkbuf[j, pl.ds(i * 16, 16)]
        if shift:
            w = lax.shift_right_logical(v, 16)
        else:
            w = lax.bitwise_and(v, jnp.full((16,), 0xFFFF, jnp.int32))
        ring[b, pl.ds(i * 16, 16)] = w


def _edge_pass_body(mode, x_hbm, pkT, sval_hbm, zmat, zvec,
                    feat_out, scal_out,
                    pkbuf, srcring, dstring, rows, svbuf, onesbuf, acc, sacc,
                    gsem, ssem, sgsem, sssem):
    c = lax.axis_index("c")
    s = lax.axis_index("s")
    r0 = s * RPS
    pltpu.sync_copy(zmat.at[pl.ds(r0, RPS)], acc.at[pl.ds(r0, RPS)])
    if mode:
        pltpu.sync_copy(zvec.at[pl.ds(r0, RPS)], sacc.at[pl.ds(r0, RPS)])
    if mode == 1:
        for i in range(CH // 16):
            onesbuf[pl.ds(i * 16, 16)] = jnp.ones((16,), jnp.float32)
    plsc.subcore_barrier()
    base = (c * NS + s) * CPS
    pltpu.sync_copy(pkT.at[pl.ds(base, CPS)], pkbuf)

    def g_desc(b):
        return pltpu.make_async_copy(
            x_hbm.at[srcring.at[b]], rows.at[b], gsem.at[b])

    def s_desc(b):
        return pltpu.make_async_copy(
            rows.at[b], acc.at[dstring.at[b]], ssem.at[b])

    def sg_desc(b):
        return pltpu.make_async_copy(
            sval_hbm.at[srcring.at[b]], svbuf.at[b], sgsem.at[b])

    def ss_desc(b):
        src = onesbuf if mode == 1 else svbuf.at[b]
        return pltpu.make_async_copy(
            src, sacc.at[dstring.at[b]], sssem.at[b])

    for b in range(NBUF):
        _unpack_idx(pkbuf, b, srcring, b, 0)
        _unpack_idx(pkbuf, b, dstring, b, 1)
        g_desc(b).start()
        if mode == 2:
            sg_desc(b).start()

    def g_body(g, carry):
        j0 = g * NBUF
        for b in range(NBUF):
            g_desc(b).wait()
            if mode == 2:
                sg_desc(b).wait()
        for b in range(NBUF):
            pltpu.async_copy(rows.at[b], acc.at[dstring.at[b]],
                             ssem.at[b], add=True)
            if mode:
                src = onesbuf if mode == 1 else svbuf.at[b]
                pltpu.async_copy(src, sacc.at[dstring.at[b]],
                                 sssem.at[b], add=True)

        @pl.when(g < NG - 1)
        def _():
            for b in range(NBUF):
                _unpack_idx(pkbuf, j0 + NBUF + b, srcring, b, 0)
        for b in range(NBUF):
            s_desc(b).wait()
            if mode:
                ss_desc(b).wait()

        @pl.when(g < NG - 1)
        def _():
            for b in range(NBUF):
                _unpack_idx(pkbuf, j0 + NBUF + b, dstring, b, 1)
                g_desc(b).start()
                if mode == 2:
                    sg_desc(b).start()
        return carry

    lax.fori_loop(0, NG, g_body, 0)
    plsc.subcore_barrier()
    pltpu.sync_copy(acc.at[pl.ds(r0, RPS)], feat_out.at[c, pl.ds(r0, RPS)])
    if mode:
        pltpu.sync_copy(sacc.at[pl.ds(r0, RPS)],
                        scal_out.at[c, pl.ds(r0, RPS)])
    else:
        pltpu.sync_copy(zvec.at[pl.ds(r0, RPS)],
                        scal_out.at[c, pl.ds(r0, RPS)])


def _sc_edge_pass(xmat, sval, pkT, zmat, zvec, mode):
    mesh = plsc.VectorSubcoreMesh(core_axis_name="c", subcore_axis_name="s")
    f = pl.kernel(
        functools.partial(_edge_pass_body, mode),
        out_type=(jax.ShapeDtypeStruct((NC, N_PAD, H), jnp.float32),
                  jax.ShapeDtypeStruct((NC, N_PAD), jnp.float32)),
        mesh=mesh,
        scratch_types=[
            pltpu.VMEM((CPS, CH), jnp.int32),
            pltpu.VMEM((NBUF, CH), jnp.int32),
            pltpu.VMEM((NBUF, CH), jnp.int32),
            pltpu.VMEM((NBUF, CH, H), jnp.float32),
            pltpu.VMEM((NBUF, CH), jnp.float32),
            pltpu.VMEM((CH,), jnp.float32),
            pltpu.VMEM_SHARED((N_PAD, H), jnp.float32),
            pltpu.VMEM_SHARED((N_PAD,), jnp.float32),
            pltpu.SemaphoreType.DMA((NBUF,)),
            pltpu.SemaphoreType.DMA((NBUF,)),
            pltpu.SemaphoreType.DMA((NBUF,)),
            pltpu.SemaphoreType.DMA((NBUF,)),
        ],
    )
    return f(xmat, pkT, sval, zmat, zvec)


def _scalar_pass_body(sval_hbm, pkT, zvec,
                      scal_out,
                      pkbuf, srcring, dstring, svbuf, sacc, gsem, ssem):
    c = lax.axis_index("c")
    s = lax.axis_index("s")
    r0 = s * RPS
    pltpu.sync_copy(zvec.at[pl.ds(r0, RPS)], sacc.at[pl.ds(r0, RPS)])
    plsc.subcore_barrier()
    base = (c * NS + s) * CPS
    pltpu.sync_copy(pkT.at[pl.ds(base, CPS)], pkbuf)

    def g_desc(b):
        return pltpu.make_async_copy(
            sval_hbm.at[srcring.at[b]], svbuf.at[b], gsem.at[b])

    def s_desc(b):
        return pltpu.make_async_copy(
            svbuf.at[b], sacc.at[dstring.at[b]], ssem.at[b])

    for b in range(NBUF_S):
        _unpack_idx(pkbuf, b, srcring, b, 0)
        _unpack_idx(pkbuf, b, dstring, b, 1)
        g_desc(b).start()

    def g_body(g, carry):
        j0 = g * NBUF_S
        for b in range(NBUF_S):
            g_desc(b).wait()
        for b in range(NBUF_S):
            pltpu.async_copy(svbuf.at[b], sacc.at[dstring.at[b]],
                             ssem.at[b], add=True)

        @pl.when(g < NG_S - 1)
        def _():
            for b in range(NBUF_S):
                _unpack_idx(pkbuf, j0 + NBUF_S + b, srcring, b, 0)
        for b in range(NBUF_S):
            s_desc(b).wait()

        @pl.when(g < NG_S - 1)
        def _():
            for b in range(NBUF_S):
                _unpack_idx(pkbuf, j0 + NBUF_S + b, dstring, b, 1)
                g_desc(b).start()
        return carry

    lax.fori_loop(0, NG_S, g_body, 0)
    plsc.subcore_barrier()
    pltpu.sync_copy(sacc.at[pl.ds(r0, RPS)], scal_out.at[c, pl.ds(r0, RPS)])


def _sc_scalar_pass(sval, pkT, zvec):
    mesh = plsc.VectorSubcoreMesh(core_axis_name="c", subcore_axis_name="s")
    f = pl.kernel(
        _scalar_pass_body,
        out_type=jax.ShapeDtypeStruct((NC, N_PAD), jnp.float32),
        mesh=mesh,
        scratch_types=[
            pltpu.VMEM((CPS, CH), jnp.int32),
            pltpu.VMEM((NBUF_S, CH), jnp.int32),
            pltpu.VMEM((NBUF_S, CH), jnp.int32),
            pltpu.VMEM((NBUF_S, CH), jnp.float32),
            pltpu.VMEM_SHARED((N_PAD,), jnp.float32),
            pltpu.SemaphoreType.DMA((NBUF_S,)),
            pltpu.SemaphoreType.DMA((NBUF_S,)),
        ],
    )
    return f(sval, pkT, zvec)


def _iota16():
    return lax.broadcasted_iota(jnp.int32, (16,), 0)


def _rank_body(score_hbm, meta_hbm,
               keep_out, m_out,
               sc_v, mbx, oidxb, kchunk, mchunk):
    c = lax.axis_index("c")
    s = lax.axis_index("s")
    w = c * NS + s
    pltpu.sync_copy(score_hbm, sc_v)
    pltpu.sync_copy(meta_hbm.at[pl.ds(32 * w, 32)], mbx)
    for i in range(8):
        oidxb[0, pl.ds(i * 16, 16)] = N + _iota16() + i * 16

    for bi in range(2):
        mv = mbx[pl.ds(16 * bi, 16)]
        start = mv[0]
        end = mv[1]
        kf = mv[2].astype(jnp.float32)
        nig = (end - start + 15) // 16

        def iouter(gi, carry):
            i_off = start + gi * 16
            si = sc_v[pl.ds(i_off, 16)]
            iidx = _iota16() + i_off
            ivalid = jnp.where(iidx < end, 1.0, 0.0)

            def jgroup(jg, acc):
                j_off = start + jg * 16
                jvec = sc_v[pl.ds(j_off, 16)]
                for l in range(16):
                    sj = jvec[l]
                    jidx = j_off + l
                    gt = jnp.where(sj > si, 1.0, 0.0)
                    eq = jnp.where(sj == si, 1.0, 0.0)
                    ltj = jnp.where(jidx < iidx, 1.0, 0.0)
                    vj = jnp.where(jidx < end, 1.0, 0.0)
                    acc = acc + (gt + eq * ltj) * vj
                return acc

            acc = lax.fori_loop(0, nig, jgroup, jnp.zeros((16,), jnp.float32))
            keep_v = jnp.where(acc < kf, 1.0, 0.0) * ivalid
            oidx = jnp.where(iidx < end, iidx, N + _iota16())
            slot = lax.rem(gi, 8) * 16
            oidxb[0, pl.ds(slot, 16)] = oidx
            kchunk[pl.ds(slot, 16)] = keep_v
            mchunk[pl.ds(slot, 16)] = keep_v * si

            @pl.when(jnp.logical_or(lax.rem(gi, 8) == 7, gi == nig - 1))
            def _():
                pltpu.sync_copy(kchunk, keep_out.at[oidxb.at[0]])
                pltpu.sync_copy(mchunk, m_out.at[oidxb.at[0]])
            return carry

        lax.fori_loop(0, nig, iouter, 0)


def _sc_rank(score_pad, meta):
    mesh = plsc.VectorSubcoreMesh(core_axis_name="c", subcore_axis_name="s")
    f = pl.kernel(
        _rank_body,
        out_type=(jax.ShapeDtypeStruct((N_PAD,), jnp.float32),
                  jax.ShapeDtypeStruct((N_PAD,), jnp.float32)),
        mesh=mesh,
        scratch_types=[
            pltpu.VMEM((N_PAD,), jnp.float32),
            pltpu.VMEM((32,), jnp.int32),
            pltpu.VMEM((1, CH), jnp.int32),
            pltpu.VMEM((CH,), jnp.float32),
            pltpu.VMEM((CH,), jnp.float32),
        ],
    )
    return f(score_pad, meta)


def _readout_body(h_ref, m_ref, k_ref, batch_ref, f1w_ref, f1b_ref, f2w_ref,
                  f2b_ref, out_ref):
    h = h_ref[...]
    m = m_ref[...]
    kf = k_ref[...]
    bvec = batch_ref[...]
    oh = jnp.equal(bvec, lax.broadcasted_iota(jnp.int32, (B, N), 0))
    wvals = jnp.where(oh, m, jnp.zeros((B, N), jnp.float32))
    s = jnp.dot(wvals, h, preferred_element_type=jnp.float32)
    cnt = jnp.sum(jnp.where(oh, kf, jnp.zeros((B, N), jnp.float32)),
                  axis=1, keepdims=True)
    g = s / jnp.maximum(cnt, 1.0)
    z1 = jnp.maximum(
        lax.dot_general(g, f1w_ref[...], (((1,), (1,)), ((), ())),
                        preferred_element_type=jnp.float32)
        + f1b_ref[...], 0.0)
    z = (lax.dot_general(z1, f2w_ref[...], (((1,), (1,)), ((), ())),
                         preferred_element_type=jnp.float32)
         + f2b_ref[...])
    zmax = jnp.max(z, axis=1, keepdims=True)
    lse = jnp.log(jnp.sum(jnp.exp(z - zmax), axis=1, keepdims=True)) + zmax
    out_ref[...] = z - lse


def _readout(h, m, keep, batch, f1w, f1b, f2w, f2b):
    return pl.pallas_call(
        _readout_body,
        out_shape=jax.ShapeDtypeStruct((B, 2), jnp.float32),
    )(h, m.reshape(1, N), keep.reshape(1, N), batch.reshape(1, N),
      f1w, f1b.reshape(1, 64), f2w, f2b.reshape(1, 2))


def kernel(x, edge_index, batch, w1l, b1l, w1r, w2l, b2l, w2r, w3l, b3l, w3r,
           w4l, b4l, w4r, p1w, p1b, p1r, p2w, p2b, p2r, f1w, f1b, f2w, f2b):
    src = edge_index[0]
    dst = edge_index[1]
    pad_e = (NCHUNK_PAD - NCHUNK) * CH
    src_p = jnp.concatenate([src, jnp.zeros((pad_e,), jnp.int32)])
    dst_p = jnp.concatenate(
        [dst, N + (jnp.arange(pad_e, dtype=jnp.int32) % (N_PAD - N))])
    pkT = jnp.bitwise_or(jnp.left_shift(dst_p, 16),
                         src_p).reshape(NCHUNK_PAD, CH)
    zmat = jnp.zeros((N_PAD, H), jnp.float32)
    zvec = jnp.zeros((N_PAD,), jnp.float32)
    ones_n = jnp.ones((N,), jnp.float32)

    P1, degp = _sc_edge_pass(x, ones_n, pkT, zmat, zvec, 1)
    deg = jnp.clip(degp[0, :N] + degp[1, :N], 1.0)[:, None]
    h1 = jax.nn.relu((P1[0, :N] + P1[1, :N]) / deg @ w1l.T + b1l + x @ w1r.T)
    P2, _ = _sc_edge_pass(h1, ones_n, pkT, zmat, zvec, 0)
    h2 = jax.nn.relu((P2[0, :N] + P2[1, :N]) / deg @ w2l.T + b2l + h1 @ w2r.T)
    a1 = (h2 @ p1w.T)[:, 0]
    c1 = (h2 @ p1r.T)[:, 0]
    sp1p = _sc_scalar_pass(a1, pkT, zvec)
    score1 = jnp.tanh(sp1p[0, :N] + sp1p[1, :N] + p1b[0] + c1)
    sizes = jnp.bincount(batch, length=B).astype(jnp.int32)
    starts = (jnp.cumsum(sizes) - sizes).astype(jnp.int32)
    ends = starts + sizes
    kper1 = jnp.ceil(0.5 * sizes.astype(jnp.float32)).astype(jnp.int32)
    pad_n = jnp.zeros((N_PAD - N,), jnp.float32)
    meta1 = jnp.pad(jnp.stack([starts, ends, kper1], axis=1),
                    ((0, 0), (0, 13))).reshape(-1)
    keep1_p, m1_p = _sc_rank(jnp.concatenate([score1, pad_n]), meta1)
    keep1 = keep1_p[:N]
    m1 = m1_p[:N]
    x3 = h2 * m1[:, None]
    P3, kdegp = _sc_edge_pass(x3, keep1, pkT, zmat, zvec, 2)
    kdeg = jnp.clip(kdegp[0, :N] + kdegp[1, :N], 1.0)[:, None]
    h3 = jax.nn.relu(
        (keep1[:, None] * (P3[0, :N] + P3[1, :N])) / kdeg @ w3l.T + b3l
        + x3 @ w3r.T
    ) * keep1[:, None]
    P4, _ = _sc_edge_pass(h3, ones_n, pkT, zmat, zvec, 0)
    h4 = jax.nn.relu(
        (keep1[:, None] * (P4[0, :N] + P4[1, :N])) / kdeg @ w4l.T + b4l
        + h3 @ w4r.T)
    az = (h4 @ p2w.T)[:, 0] * keep1
    c2 = (h4 @ p2r.T)[:, 0]
    sp2p = _sc_scalar_pass(az, pkT, zvec)
    score2 = jnp.tanh(keep1 * (sp2p[0, :N] + sp2p[1, :N]) + p2b[0] + c2)
    score_m2 = jnp.where(keep1 > 0, score2, -2.0)
    counts2 = jax.ops.segment_sum(keep1, batch, num_segments=B)
    kper2 = jnp.ceil(0.5 * counts2).astype(jnp.int32)
    key = batch.astype(jnp.float32) * 8.0 - score_m2
    order = jnp.argsort(key)
    bs = batch[order]
    rank = jnp.arange(N) - starts[bs]
    keep_sorted = rank < kper2[bs]
    keep2 = (jnp.zeros((N,), bool).at[order].set(keep_sorted)
             ).astype(jnp.float32) * keep1
    m2 = score_m2 * keep2
    return _readout(h4, m2, keep2, batch, f1w, f1b, f2w, f2b)
